# baseline (speedup 1.0000x reference)
import functools

import jax
import jax.numpy as jnp
from jax.experimental import pallas as pl
from jax.experimental.pallas import tpu as pltpu
from jax.experimental.pallas import tpu_sc as plsc

TAU_TR = 2.0
TAU_RE = 1.5
BETA_TR = 0.25
BETA_RE = 0.25
LAMBDA_COUPLE = 0.1
SPARSITY_W = 0.1
EPS = 1e-10

BR_Q = 128
BR_C = 64


def _nt(a, b):
    return jax.lax.dot_general(a, b, (((1,), (1,)), ((), ())),
                               preferred_element_type=jnp.float32)


def _nn(a, b):
    return jax.lax.dot_general(a, b, (((1,), (0,)), ((), ())),
                               preferred_element_type=jnp.float32)


def _quantize_body(tau, h_ref, cb_ref, hsq_ref, cbsq_ref, u_ref,
                   q_ref, soft_ref, idx_ref, acc_ref):
    step = pl.program_id(0)

    @pl.when(step == 0)
    def _init():
        acc_ref[...] = jnp.zeros_like(acc_ref)

    h = h_ref[...]
    cb = cb_ref[...]
    k = cb.shape[0]
    distances = (hsq_ref[...] + cbsq_ref[...]) - 2.0 * _nt(h, cb)
    g = -jnp.log(-jnp.log(u_ref[...] + EPS) + EPS)
    x = (-distances + g) / tau
    x = x - jnp.max(x, axis=1, keepdims=True)
    ex = jnp.exp(x)
    q = ex / jnp.sum(ex, axis=1, keepdims=True)
    q_ref[...] = q
    c_tilde = _nn(q, cb)
    soft_ref[...] = c_tilde
    qmax = jnp.max(q, axis=1, keepdims=True)
    iota = jax.lax.broadcasted_iota(jnp.int32, (q.shape[0], k), 1)
    idx = jnp.min(jnp.where(q == qmax, iota, k), axis=1).astype(jnp.int32)
    idx_ref[0, 0, :] = idx
    mse_sum = jnp.sum((c_tilde - h) ** 2)
    lane = jax.lax.broadcasted_iota(jnp.int32, (1, 128), 1)
    acc_ref[...] += jnp.where(lane == 0, mse_sum, 0.0)


def _quantize_couple_body(tau, h_ref, cb_ref, hsq_ref, cbsq_ref, u_ref,
                          w1_ref, b1_ref, w2_ref, b2_ref, q_tr_ref,
                          q_ref, soft_ref, idx_ref, acc_ref):
    step = pl.program_id(0)

    @pl.when(step == 0)
    def _init():
        acc_ref[...] = jnp.zeros_like(acc_ref)

    h = h_ref[...]
    cb = cb_ref[...]
    k = cb.shape[0]
    distances = (hsq_ref[...] + cbsq_ref[...]) - 2.0 * _nt(h, cb)
    g = -jnp.log(-jnp.log(u_ref[...] + EPS) + EPS)
    x = (-distances + g) / tau
    x = x - jnp.max(x, axis=1, keepdims=True)
    ex = jnp.exp(x)
    q = ex / jnp.sum(ex, axis=1, keepdims=True)
    q_ref[...] = q
    c_tilde = _nn(q, cb)
    soft_ref[...] = c_tilde
    qmax = jnp.max(q, axis=1, keepdims=True)
    iota = jax.lax.broadcasted_iota(jnp.int32, (q.shape[0], k), 1)
    idx = jnp.min(jnp.where(q == qmax, iota, k), axis=1).astype(jnp.int32)
    idx_ref[0, 0, :] = idx
    mse_sum = jnp.sum((c_tilde - h) ** 2)

    hdn = _nt(q.astype(jnp.bfloat16), w1_ref[...]) + b1_ref[...]
    hdn = hdn * jax.nn.sigmoid(hdn)
    x2 = (_nt(hdn.astype(jnp.bfloat16), w2_ref[...]) + b2_ref[...]) / TAU_TR
    m = jnp.max(x2, axis=1, keepdims=True)
    lse = jnp.log(jnp.sum(jnp.exp(x2 - m), axis=1, keepdims=True))
    tgt = q_tr_ref[...]
    t_entropy = jnp.sum(tgt * jnp.log(jnp.maximum(tgt, 1e-30)), axis=1,
                        keepdims=True)
    cross = jnp.sum(tgt * x2, axis=1, keepdims=True)
    tgt_sum = jnp.sum(tgt, axis=1, keepdims=True)
    kl_sum = jnp.sum(t_entropy - cross + (m + lse) * tgt_sum)

    lane = jax.lax.broadcasted_iota(jnp.int32, (1, 128), 1)
    acc_ref[...] += (jnp.where(lane == 0, mse_sum, 0.0)
                     + jnp.where(lane == 1, kl_sum, 0.0))


def _quantize(h, cb, h_sq, cb_sq, g, tau):
    b, d = h.shape
    k = cb.shape[0]
    grid = b // BR_Q
    full = lambda shape: pl.BlockSpec(shape, lambda i: (0,) * len(shape))
    rows = lambda shape: pl.BlockSpec(
        shape, lambda i: (i,) + (0,) * (len(shape) - 1))
    return pl.pallas_call(
        functools.partial(_quantize_body, tau),
        grid=(grid,),
        in_specs=[rows((BR_Q, d)), full((k, d)), rows((BR_Q, 1)),
                  full((1, k)), rows((BR_Q, k))],
        out_specs=[rows((BR_Q, k)), rows((BR_Q, d)),
                   rows((1, 1, BR_Q)), full((1, 128))],
        out_shape=(
            jax.ShapeDtypeStruct((b, k), jnp.float32),
            jax.ShapeDtypeStruct((b, d), jnp.float32),
            jax.ShapeDtypeStruct((grid, 1, BR_Q), jnp.int32),
            jax.ShapeDtypeStruct((1, 128), jnp.float32),
        ),
        compiler_params=pltpu.CompilerParams(
            dimension_semantics=("arbitrary",),
        ),
    )(h, cb, h_sq, cb_sq, g)


def _hard_gather(cb_tr, idx_tr, cb_re, idx_re):
    b = idx_tr.shape[0]
    d = cb_tr.shape[1]
    info = plsc.get_sparse_core_info()
    nc = info.num_cores
    nw = nc * info.num_subcores
    b_per_w = b // nw
    mesh = plsc.VectorSubcoreMesh(core_axis_name="c", subcore_axis_name="s")

    @functools.partial(
        pl.kernel, mesh=mesh,
        out_type=(jax.ShapeDtypeStruct((b, d), jnp.float32),
                  jax.ShapeDtypeStruct((b, d), jnp.float32)),
        scratch_types=[pltpu.VMEM((b_per_w,), jnp.int32),
                       pltpu.VMEM((b_per_w, d), jnp.float32),
                       pltpu.SemaphoreType.DMA],
    )
    def gather_k(cb_tr_hbm, idx_tr_hbm, cb_re_hbm, idx_re_hbm,
                 out_tr_hbm, out_re_hbm, idx_v, rows_v, sem):
        wid = jax.lax.axis_index("s") * nc + jax.lax.axis_index("c")
        base = wid * b_per_w
        pltpu.sync_copy(idx_tr_hbm.at[pl.ds(base, b_per_w)], idx_v)
        pltpu.async_copy(cb_tr_hbm.at[idx_v], rows_v, sem).wait()
        pltpu.sync_copy(rows_v, out_tr_hbm.at[pl.ds(base, b_per_w)])
        pltpu.sync_copy(idx_re_hbm.at[pl.ds(base, b_per_w)], idx_v)
        pltpu.async_copy(cb_re_hbm.at[idx_v], rows_v, sem).wait()
        pltpu.sync_copy(rows_v, out_re_hbm.at[pl.ds(base, b_per_w)])

    return gather_k(cb_tr, idx_tr, cb_re, idx_re)


def _quantize_couple(h, cb, h_sq, cb_sq, u, tau, W1, b1, W2, b2, q_tr):
    b, d = h.shape
    k = cb.shape[0]
    k_tr = W2.shape[0]
    hid = W1.shape[0]
    grid = b // BR_C
    full = lambda shape: pl.BlockSpec(shape, lambda i: (0,) * len(shape))
    rows = lambda shape: pl.BlockSpec(
        shape, lambda i: (i,) + (0,) * (len(shape) - 1))
    return pl.pallas_call(
        functools.partial(_quantize_couple_body, tau),
        grid=(grid,),
        in_specs=[rows((BR_C, d)), full((k, d)), rows((BR_C, 1)),
                  full((1, k)), rows((BR_C, k)),
                  full((hid, k)), full((1, hid)),
                  full((k_tr, hid)), full((1, k_tr)), rows((BR_C, k_tr))],
        out_specs=[rows((BR_C, k)), rows((BR_C, d)),
                   rows((1, 1, BR_C)), full((1, 128))],
        out_shape=(
            jax.ShapeDtypeStruct((b, k), jnp.float32),
            jax.ShapeDtypeStruct((b, d), jnp.float32),
            jax.ShapeDtypeStruct((grid, 1, BR_C), jnp.int32),
            jax.ShapeDtypeStruct((1, 128), jnp.float32),
        ),
        compiler_params=pltpu.CompilerParams(
            dimension_semantics=("arbitrary",),
        ),
    )(h, cb, h_sq, cb_sq, u, W1.astype(jnp.bfloat16), b1.reshape(1, hid),
      W2.astype(jnp.bfloat16), b2.reshape(1, k_tr), q_tr)


@jax.jit
def kernel(h_tr, h_re, codebook_tr, codebook_re, W1, b1, W2, b2,
           sparsity_mask, u_tr, u_re):
    b, d = h_tr.shape

    hsq_tr = jnp.sum(h_tr ** 2, axis=1, keepdims=True)
    hsq_re = jnp.sum(h_re ** 2, axis=1, keepdims=True)
    cbsq_tr = jnp.sum(codebook_tr ** 2, axis=1).reshape(1, -1)
    cbsq_re = jnp.sum(codebook_re ** 2, axis=1).reshape(1, -1)

    q_tr, soft_tr, idx_tr, acc_tr = _quantize(
        h_tr, codebook_tr, hsq_tr, cbsq_tr, u_tr, TAU_TR)
    q_re, soft_re, idx_re, acc_re = _quantize_couple(
        h_re, codebook_re, hsq_re, cbsq_re, u_re, TAU_RE, W1, b1, W2, b2,
        q_tr)
    hard_tr, hard_re = _hard_gather(
        codebook_tr, idx_tr.reshape(-1), codebook_re, idx_re.reshape(-1))

    mse_tr = acc_tr[0, 0] / (b * d)
    mse_re = acc_re[0, 0] / (b * d)
    loss_tr = mse_tr + BETA_TR * mse_tr
    loss_re = mse_re + BETA_RE * mse_re
    kl = acc_re[0, 1] / b
    coupling_loss = kl * LAMBDA_COUPLE
    sparsity_loss = jnp.mean(jnp.abs(sparsity_mask))
    total_loss = (loss_tr + loss_re) + coupling_loss + SPARSITY_W * sparsity_loss

    quant_tr = soft_tr + (hard_tr - soft_tr)
    quant_re = soft_re + (hard_re - soft_re)
    return (q_tr, soft_tr, hard_tr, quant_tr, q_re, soft_re, hard_re,
            quant_re, coupling_loss, total_loss)


# speedup vs baseline: 1.4721x; 1.4721x over previous
import functools

import jax
import jax.numpy as jnp
from jax.experimental import pallas as pl
from jax.experimental.pallas import tpu as pltpu
from jax.experimental.pallas import tpu_sc as plsc

TAU_TR = 2.0
TAU_RE = 1.5
BETA_TR = 0.25
BETA_RE = 0.25
LAMBDA_COUPLE = 0.1
SPARSITY_W = 0.1
EPS = 1e-10

BR_Q = 128
BR_C = 256


def _nt(a, b):
    return jax.lax.dot_general(a, b, (((1,), (1,)), ((), ())),
                               preferred_element_type=jnp.float32)


def _nn(a, b):
    return jax.lax.dot_general(a, b, (((1,), (0,)), ((), ())),
                               preferred_element_type=jnp.float32)


def _quantize_body(tau, h_ref, cb_ref, hsq_ref, cbsq_ref, u_ref,
                   q_ref, soft_ref, idx_ref, acc_ref):
    step = pl.program_id(0)

    @pl.when(step == 0)
    def _init():
        acc_ref[...] = jnp.zeros_like(acc_ref)

    h = h_ref[...]
    cb = cb_ref[...]
    k = cb.shape[0]
    distances = (hsq_ref[...] + cbsq_ref[...]) - 2.0 * _nt(h, cb)
    g = -jnp.log(-jnp.log(u_ref[...] + EPS) + EPS)
    x = (-distances + g) / tau
    x = x - jnp.max(x, axis=1, keepdims=True)
    ex = jnp.exp(x)
    q = ex / jnp.sum(ex, axis=1, keepdims=True)
    q_ref[...] = q
    c_tilde = _nn(q, cb)
    soft_ref[...] = c_tilde
    qmax = jnp.max(q, axis=1, keepdims=True)
    iota = jax.lax.broadcasted_iota(jnp.int32, (q.shape[0], k), 1)
    idx = jnp.min(jnp.where(q == qmax, iota, k), axis=1).astype(jnp.int32)
    idx_ref[0, 0, :] = idx
    mse_sum = jnp.sum((c_tilde - h) ** 2)
    lane = jax.lax.broadcasted_iota(jnp.int32, (1, 128), 1)
    acc_ref[...] += jnp.where(lane == 0, mse_sum, 0.0)


def _quantize_hdn_body(tau, h_ref, cb_ref, hsq_ref, cbsq_ref, u_ref,
                       w1_ref, b1_ref,
                       q_ref, soft_ref, idx_ref, hdn_ref, acc_ref):
    step = pl.program_id(0)

    @pl.when(step == 0)
    def _init():
        acc_ref[...] = jnp.zeros_like(acc_ref)

    h = h_ref[...]
    cb = cb_ref[...]
    k = cb.shape[0]
    distances = (hsq_ref[...] + cbsq_ref[...]) - 2.0 * _nt(h, cb)
    g = -jnp.log(-jnp.log(u_ref[...] + EPS) + EPS)
    x = (-distances + g) / tau
    x = x - jnp.max(x, axis=1, keepdims=True)
    ex = jnp.exp(x)
    q = ex / jnp.sum(ex, axis=1, keepdims=True)
    q_ref[...] = q
    c_tilde = _nn(q, cb)
    soft_ref[...] = c_tilde
    qmax = jnp.max(q, axis=1, keepdims=True)
    iota = jax.lax.broadcasted_iota(jnp.int32, (q.shape[0], k), 1)
    idx = jnp.min(jnp.where(q == qmax, iota, k), axis=1).astype(jnp.int32)
    idx_ref[0, 0, :] = idx
    mse_sum = jnp.sum((c_tilde - h) ** 2)

    hdn = _nt(q.astype(jnp.bfloat16), w1_ref[...]) + b1_ref[...]
    hdn_ref[...] = hdn * jax.nn.sigmoid(hdn)

    lane = jax.lax.broadcasted_iota(jnp.int32, (1, 128), 1)
    acc_ref[...] += jnp.where(lane == 0, mse_sum, 0.0)


def _kl_body(hdn_ref, w2_ref, b2_ref, q_tr_ref, acc_ref):
    step = pl.program_id(0)

    @pl.when(step == 0)
    def _init():
        acc_ref[...] = jnp.zeros_like(acc_ref)

    x2 = (_nt(hdn_ref[...].astype(jnp.bfloat16), w2_ref[...])
          + b2_ref[...]) / TAU_TR
    m = jnp.max(x2, axis=1, keepdims=True)
    lse = jnp.log(jnp.sum(jnp.exp(x2 - m), axis=1, keepdims=True))
    tgt = q_tr_ref[...]
    t_entropy = jnp.sum(tgt * jnp.log(jnp.maximum(tgt, 1e-30)), axis=1,
                        keepdims=True)
    cross = jnp.sum(tgt * x2, axis=1, keepdims=True)
    tgt_sum = jnp.sum(tgt, axis=1, keepdims=True)
    kl_sum = jnp.sum(t_entropy - cross + (m + lse) * tgt_sum)
    lane = jax.lax.broadcasted_iota(jnp.int32, (1, 128), 1)
    acc_ref[...] += jnp.where(lane == 0, kl_sum, 0.0)


def _quantize(h, cb, h_sq, cb_sq, g, tau):
    b, d = h.shape
    k = cb.shape[0]
    grid = b // BR_Q
    full = lambda shape: pl.BlockSpec(shape, lambda i: (0,) * len(shape))
    rows = lambda shape: pl.BlockSpec(
        shape, lambda i: (i,) + (0,) * (len(shape) - 1))
    return pl.pallas_call(
        functools.partial(_quantize_body, tau),
        grid=(grid,),
        in_specs=[rows((BR_Q, d)), full((k, d)), rows((BR_Q, 1)),
                  full((1, k)), rows((BR_Q, k))],
        out_specs=[rows((BR_Q, k)), rows((BR_Q, d)),
                   rows((1, 1, BR_Q)), full((1, 128))],
        out_shape=(
            jax.ShapeDtypeStruct((b, k), jnp.float32),
            jax.ShapeDtypeStruct((b, d), jnp.float32),
            jax.ShapeDtypeStruct((grid, 1, BR_Q), jnp.int32),
            jax.ShapeDtypeStruct((1, 128), jnp.float32),
        ),
        compiler_params=pltpu.CompilerParams(
            dimension_semantics=("arbitrary",),
        ),
    )(h, cb, h_sq, cb_sq, g)


def _hard_gather(cb_tr, idx_tr, cb_re, idx_re):
    b = idx_tr.shape[0]
    d = cb_tr.shape[1]
    info = plsc.get_sparse_core_info()
    nc = info.num_cores
    nw = nc * info.num_subcores
    b_per_w = b // nw
    mesh = plsc.VectorSubcoreMesh(core_axis_name="c", subcore_axis_name="s")

    @functools.partial(
        pl.kernel, mesh=mesh,
        out_type=(jax.ShapeDtypeStruct((b, d), jnp.float32),
                  jax.ShapeDtypeStruct((b, d), jnp.float32)),
        scratch_types=[pltpu.VMEM((b_per_w,), jnp.int32),
                       pltpu.VMEM((b_per_w, d), jnp.float32),
                       pltpu.SemaphoreType.DMA],
    )
    def gather_k(cb_tr_hbm, idx_tr_hbm, cb_re_hbm, idx_re_hbm,
                 out_tr_hbm, out_re_hbm, idx_v, rows_v, sem):
        wid = jax.lax.axis_index("s") * nc + jax.lax.axis_index("c")
        base = wid * b_per_w
        pltpu.sync_copy(idx_tr_hbm.at[pl.ds(base, b_per_w)], idx_v)
        pltpu.async_copy(cb_tr_hbm.at[idx_v], rows_v, sem).wait()
        pltpu.sync_copy(rows_v, out_tr_hbm.at[pl.ds(base, b_per_w)])
        pltpu.sync_copy(idx_re_hbm.at[pl.ds(base, b_per_w)], idx_v)
        pltpu.async_copy(cb_re_hbm.at[idx_v], rows_v, sem).wait()
        pltpu.sync_copy(rows_v, out_re_hbm.at[pl.ds(base, b_per_w)])

    return gather_k(cb_tr, idx_tr, cb_re, idx_re)


def _quantize_hdn(h, cb, h_sq, cb_sq, u, tau, W1, b1):
    b, d = h.shape
    k = cb.shape[0]
    hid = W1.shape[0]
    grid = b // BR_Q
    full = lambda shape: pl.BlockSpec(shape, lambda i: (0,) * len(shape))
    rows = lambda shape: pl.BlockSpec(
        shape, lambda i: (i,) + (0,) * (len(shape) - 1))
    return pl.pallas_call(
        functools.partial(_quantize_hdn_body, tau),
        grid=(grid,),
        in_specs=[rows((BR_Q, d)), full((k, d)), rows((BR_Q, 1)),
                  full((1, k)), rows((BR_Q, k)),
                  full((hid, k)), full((1, hid))],
        out_specs=[rows((BR_Q, k)), rows((BR_Q, d)),
                   rows((1, 1, BR_Q)), rows((BR_Q, hid)), full((1, 128))],
        out_shape=(
            jax.ShapeDtypeStruct((b, k), jnp.float32),
            jax.ShapeDtypeStruct((b, d), jnp.float32),
            jax.ShapeDtypeStruct((grid, 1, BR_Q), jnp.int32),
            jax.ShapeDtypeStruct((b, hid), jnp.float32),
            jax.ShapeDtypeStruct((1, 128), jnp.float32),
        ),
        compiler_params=pltpu.CompilerParams(
            dimension_semantics=("arbitrary",),
        ),
    )(h, cb, h_sq, cb_sq, u, W1.astype(jnp.bfloat16), b1.reshape(1, hid))


def _kl(hdn, W2, b2, q_tr):
    b, hid = hdn.shape
    k_tr = W2.shape[0]
    grid = b // BR_C
    full = lambda shape: pl.BlockSpec(shape, lambda i: (0,) * len(shape))
    rows = lambda shape: pl.BlockSpec(
        shape, lambda i: (i,) + (0,) * (len(shape) - 1))
    return pl.pallas_call(
        _kl_body,
        grid=(grid,),
        in_specs=[rows((BR_C, hid)), full((k_tr, hid)), full((1, k_tr)),
                  rows((BR_C, k_tr))],
        out_specs=[full((1, 128))],
        out_shape=(jax.ShapeDtypeStruct((1, 128), jnp.float32),),
        compiler_params=pltpu.CompilerParams(
            dimension_semantics=("arbitrary",),
        ),
    )(hdn, W2.astype(jnp.bfloat16), b2.reshape(1, k_tr), q_tr)


@jax.jit
def kernel(h_tr, h_re, codebook_tr, codebook_re, W1, b1, W2, b2,
           sparsity_mask, u_tr, u_re):
    b, d = h_tr.shape

    hsq_tr = jnp.sum(h_tr ** 2, axis=1, keepdims=True)
    hsq_re = jnp.sum(h_re ** 2, axis=1, keepdims=True)
    cbsq_tr = jnp.sum(codebook_tr ** 2, axis=1).reshape(1, -1)
    cbsq_re = jnp.sum(codebook_re ** 2, axis=1).reshape(1, -1)

    q_tr, soft_tr, idx_tr, acc_tr = _quantize(
        h_tr, codebook_tr, hsq_tr, cbsq_tr, u_tr, TAU_TR)
    q_re, soft_re, idx_re, hdn, acc_re = _quantize_hdn(
        h_re, codebook_re, hsq_re, cbsq_re, u_re, TAU_RE, W1, b1)
    hard_tr, hard_re = _hard_gather(
        codebook_tr, idx_tr.reshape(-1), codebook_re, idx_re.reshape(-1))
    (acc_kl,) = _kl(hdn, W2, b2, q_tr)

    mse_tr = acc_tr[0, 0] / (b * d)
    mse_re = acc_re[0, 0] / (b * d)
    loss_tr = mse_tr + BETA_TR * mse_tr
    loss_re = mse_re + BETA_RE * mse_re
    kl = acc_kl[0, 0] / b
    coupling_loss = kl * LAMBDA_COUPLE
    sparsity_loss = jnp.mean(jnp.abs(sparsity_mask))
    total_loss = (loss_tr + loss_re) + coupling_loss + SPARSITY_W * sparsity_loss

    quant_tr = soft_tr + (hard_tr - soft_tr)
    quant_re = soft_re + (hard_re - soft_re)
    return (q_tr, soft_tr, hard_tr, quant_tr, q_re, soft_re, hard_re,
            quant_re, coupling_loss, total_loss)
